# baseline (speedup 1.0000x reference)
import functools

import jax
import jax.numpy as jnp
from jax import lax
from jax.experimental import pallas as pl
from jax.experimental.pallas import tpu as pltpu
from jax.experimental.pallas import tpu_sc as plsc

_K = 16
_HALF = 160
_NBANK = 3
_SC_B = 10240
_TC_BLK = 512


def _sc_gather_call(V, D, B):
    info = plsc.get_sparse_core_info()
    NC, NS = info.num_cores, info.num_subcores
    NW = NC * NS
    b_per_w = B // NW
    n_half = b_per_w // _HALF
    n_chunk = _HALF // _K
    mesh = plsc.VectorSubcoreMesh(core_axis_name="c", subcore_axis_name="s")

    @functools.partial(
        pl.kernel,
        mesh=mesh,
        out_type=jax.ShapeDtypeStruct((B, D), jnp.float32),
        scratch_types=[
            pltpu.VMEM((b_per_w,), jnp.int32),
            pltpu.VMEM((_NBANK, _K, D, 128), jnp.float32),
            pltpu.VMEM((_HALF, D), jnp.float32),
            pltpu.SemaphoreType.DMA,
            pltpu.SemaphoreType.DMA,
            pltpu.SemaphoreType.DMA,
        ],
        compiler_params=pltpu.CompilerParams(needs_layout_passes=False),
    )
    def k(table_hbm, idx_hbm, out_hbm, idx_v, ring, out_v, sem0, sem1, sem2):
        wid = lax.axis_index("s") * NC + lax.axis_index("c")
        base = wid * b_per_w
        pltpu.sync_copy(idx_hbm.at[pl.ds(base, b_per_w)], idx_v)

        lane = lax.iota(jnp.int32, 16)
        neg_inf = jnp.int32(-2147483648)
        sems = [sem0, sem1, sem2]
        n_chunks_total = n_half * n_chunk

        def scalar_at(ci, j):
            iv16 = idx_v[pl.ds(ci * _K, _K)]
            return jnp.max(jnp.where(lane == j, iv16, neg_inf))

        def fire(ci, bank):
            def body(j, _):
                r = scalar_at(ci, j)
                blk = pl.multiple_of(
                    jnp.bitwise_and(r, jnp.int32(~127)), 128
                )
                pltpu.async_copy(
                    table_hbm.at[:, pl.ds(blk, 128)],
                    ring.at[bank, j],
                    sems[bank],
                )
                return 0

            lax.fori_loop(0, _K, body, 0)

        def drain_extract(ci, lc, bank):
            def wait_body(j, _):
                pltpu.make_async_copy(
                    table_hbm.at[:, pl.ds(0, 128)],
                    ring.at[bank, j],
                    sems[bank],
                ).wait()
                return 0

            lax.fori_loop(0, _K, wait_body, 0)

            def ext_body(j, _):
                r = scalar_at(ci, j)
                sub = jnp.bitwise_and(r, jnp.int32(127))
                val = plsc.load_gather(
                    ring.at[bank, j], [lane, jnp.full((16,), sub, jnp.int32)]
                )
                row = lc * _K + j
                plsc.store_scatter(
                    out_v, [jnp.full((16,), row, jnp.int32), lane], val
                )
                return 0

            lax.fori_loop(0, _K, ext_body, 0)

        fire(jnp.int32(0), 0)
        fire(jnp.int32(1), 1 % _NBANK)
        for ci in range(n_chunks_total):
            if ci + 2 < n_chunks_total:
                fire(jnp.int32(ci + 2), (ci + 2) % _NBANK)
            drain_extract(jnp.int32(ci), jnp.int32(ci % n_chunk), ci % _NBANK)
            if (ci + 1) % n_chunk == 0:
                half = ci // n_chunk
                pltpu.sync_copy(
                    out_v, out_hbm.at[pl.ds(base + half * _HALF, _HALF)]
                )

    return k


def _tc_gather_call(V, D, B):
    n_steps = B // _TC_BLK

    def body(idx_s, table_ref, out_ref, slab, sem):
        g = pl.program_id(0)
        base = g * _TC_BLK
        for i in range(_TC_BLK):
            r = idx_s[base + i]
            blk = pl.multiple_of(jnp.bitwise_and(r, jnp.int32(~127)), 128)
            pltpu.make_async_copy(
                table_ref.at[:, pl.ds(blk, 128)], slab.at[i], sem
            ).start()
        for i in range(_TC_BLK):
            pltpu.make_async_copy(
                table_ref.at[:, pl.ds(0, 128)], slab.at[i], sem
            ).wait()
        lanes = lax.broadcasted_iota(jnp.int32, (D, 128), 1)
        for q in range(_TC_BLK // 128):
            acc = jnp.zeros((D, 128), jnp.float32)
            for li in range(128):
                i = q * 128 + li
                r = idx_s[base + i]
                sub = jnp.bitwise_and(r, jnp.int32(127))
                rolled = pltpu.roll(slab[i], jnp.int32(li) - sub, axis=1)
                acc = jnp.where(lanes == li, rolled, acc)
            out_ref[:, q * 128 : (q + 1) * 128] = acc

    grid_spec = pltpu.PrefetchScalarGridSpec(
        num_scalar_prefetch=1,
        grid=(n_steps,),
        in_specs=[pl.BlockSpec(memory_space=pl.ANY)],
        out_specs=pl.BlockSpec((D, _TC_BLK), lambda g, idx_s: (0, g)),
        scratch_shapes=[
            pltpu.VMEM((_TC_BLK, D, 128), jnp.float32),
            pltpu.SemaphoreType.DMA,
        ],
    )
    return pl.pallas_call(
        body,
        grid_spec=grid_spec,
        out_shape=jax.ShapeDtypeStruct((D, B), jnp.float32),
    )


def kernel(data, channelindex):
    V, D = data.shape
    (B,) = channelindex.shape
    table_t = data.T
    idx = channelindex.astype(jnp.int32)
    out_sc = _sc_gather_call(V, D, _SC_B)(table_t, idx[:_SC_B])
    out_tc_t = _tc_gather_call(V, D, B - _SC_B)(idx[_SC_B:], table_t)
    return jnp.concatenate([out_sc, out_tc_t.T], axis=0)


# speedup vs baseline: 1.0350x; 1.0350x over previous
import functools

import jax
import jax.numpy as jnp
from jax import lax
from jax.experimental import pallas as pl
from jax.experimental.pallas import tpu as pltpu
from jax.experimental.pallas import tpu_sc as plsc

_K = 16
_HALF = 208
_NBANK = 3
_SC_B = 13312
_TC_BLK = 512


def _sc_gather_call(V, D, B):
    info = plsc.get_sparse_core_info()
    NC, NS = info.num_cores, info.num_subcores
    NW = NC * NS
    b_per_w = B // NW
    n_half = b_per_w // _HALF
    n_chunk = _HALF // _K
    mesh = plsc.VectorSubcoreMesh(core_axis_name="c", subcore_axis_name="s")

    @functools.partial(
        pl.kernel,
        mesh=mesh,
        out_type=jax.ShapeDtypeStruct((B, D), jnp.float32),
        scratch_types=[
            pltpu.VMEM((b_per_w,), jnp.int32),
            pltpu.VMEM((_NBANK, _K, D, 128), jnp.float32),
            pltpu.VMEM((_HALF, D), jnp.float32),
            pltpu.SemaphoreType.DMA,
            pltpu.SemaphoreType.DMA,
            pltpu.SemaphoreType.DMA,
        ],
        compiler_params=pltpu.CompilerParams(needs_layout_passes=False),
    )
    def k(table_hbm, idx_hbm, out_hbm, idx_v, ring, out_v, sem0, sem1, sem2):
        wid = lax.axis_index("s") * NC + lax.axis_index("c")
        base = wid * b_per_w
        pltpu.sync_copy(idx_hbm.at[pl.ds(base, b_per_w)], idx_v)

        lane = lax.iota(jnp.int32, 16)
        neg_inf = jnp.int32(-2147483648)
        sems = [sem0, sem1, sem2]
        n_chunks_total = n_half * n_chunk

        def scalar_at(ci, j):
            iv16 = idx_v[pl.ds(ci * _K, _K)]
            return jnp.max(jnp.where(lane == j, iv16, neg_inf))

        def fire(ci, bank):
            def body(j, _):
                r = scalar_at(ci, j)
                blk = pl.multiple_of(
                    jnp.bitwise_and(r, jnp.int32(~127)), 128
                )
                pltpu.async_copy(
                    table_hbm.at[:, pl.ds(blk, 128)],
                    ring.at[bank, j],
                    sems[bank],
                )
                return 0

            lax.fori_loop(0, _K, body, 0)

        def drain_extract(ci, lc, bank):
            def wait_body(j, _):
                pltpu.make_async_copy(
                    table_hbm.at[:, pl.ds(0, 128)],
                    ring.at[bank, j],
                    sems[bank],
                ).wait()
                return 0

            lax.fori_loop(0, _K, wait_body, 0)

            def ext_body(j, _):
                r = scalar_at(ci, j)
                sub = jnp.bitwise_and(r, jnp.int32(127))
                val = plsc.load_gather(
                    ring.at[bank, j], [lane, jnp.full((16,), sub, jnp.int32)]
                )
                row = lc * _K + j
                plsc.store_scatter(
                    out_v, [jnp.full((16,), row, jnp.int32), lane], val
                )
                return 0

            lax.fori_loop(0, _K, ext_body, 0)

        fire(jnp.int32(0), 0)
        fire(jnp.int32(1), 1 % _NBANK)
        for ci in range(n_chunks_total):
            if ci + 2 < n_chunks_total:
                fire(jnp.int32(ci + 2), (ci + 2) % _NBANK)
            drain_extract(jnp.int32(ci), jnp.int32(ci % n_chunk), ci % _NBANK)
            if (ci + 1) % n_chunk == 0:
                half = ci // n_chunk
                pltpu.sync_copy(
                    out_v, out_hbm.at[pl.ds(base + half * _HALF, _HALF)]
                )

    return k


def _tc_gather_call(V, D, B):
    n_steps = B // _TC_BLK

    def body(idx_s, table_ref, out_ref, slab, sem):
        g = pl.program_id(0)
        base = g * _TC_BLK
        for i in range(_TC_BLK):
            r = idx_s[base + i]
            blk = pl.multiple_of(jnp.bitwise_and(r, jnp.int32(~127)), 128)
            pltpu.make_async_copy(
                table_ref.at[:, pl.ds(blk, 128)], slab.at[i], sem
            ).start()
        for i in range(_TC_BLK):
            pltpu.make_async_copy(
                table_ref.at[:, pl.ds(0, 128)], slab.at[i], sem
            ).wait()
        lanes = lax.broadcasted_iota(jnp.int32, (D, 128), 1)
        for q in range(_TC_BLK // 128):
            acc = jnp.zeros((D, 128), jnp.float32)
            for li in range(128):
                i = q * 128 + li
                r = idx_s[base + i]
                sub = jnp.bitwise_and(r, jnp.int32(127))
                rolled = pltpu.roll(slab[i], jnp.int32(li) - sub, axis=1)
                acc = jnp.where(lanes == li, rolled, acc)
            out_ref[:, q * 128 : (q + 1) * 128] = acc

    grid_spec = pltpu.PrefetchScalarGridSpec(
        num_scalar_prefetch=1,
        grid=(n_steps,),
        in_specs=[pl.BlockSpec(memory_space=pl.ANY)],
        out_specs=pl.BlockSpec((D, _TC_BLK), lambda g, idx_s: (0, g)),
        scratch_shapes=[
            pltpu.VMEM((_TC_BLK, D, 128), jnp.float32),
            pltpu.SemaphoreType.DMA,
        ],
    )
    return pl.pallas_call(
        body,
        grid_spec=grid_spec,
        out_shape=jax.ShapeDtypeStruct((D, B), jnp.float32),
    )


def kernel(data, channelindex):
    V, D = data.shape
    (B,) = channelindex.shape
    table_t = data.T
    idx = channelindex.astype(jnp.int32)
    out_sc = _sc_gather_call(V, D, _SC_B)(table_t, idx[:_SC_B])
    out_tc_t = _tc_gather_call(V, D, B - _SC_B)(idx[_SC_B:], table_t)
    return jnp.concatenate([out_sc, out_tc_t.T], axis=0)


# speedup vs baseline: 1.0648x; 1.0287x over previous
import functools

import jax
import jax.numpy as jnp
from jax import lax
from jax.experimental import pallas as pl
from jax.experimental.pallas import tpu as pltpu
from jax.experimental.pallas import tpu_sc as plsc

_K = 16
_HALF = 192
_NBANK = 3
_SC_B = 12288
_TC_BLK = 512


def _sc_gather_call(V, D, B):
    info = plsc.get_sparse_core_info()
    NC, NS = info.num_cores, info.num_subcores
    NW = NC * NS
    b_per_w = B // NW
    n_half = b_per_w // _HALF
    n_chunk = _HALF // _K
    mesh = plsc.VectorSubcoreMesh(core_axis_name="c", subcore_axis_name="s")

    @functools.partial(
        pl.kernel,
        mesh=mesh,
        out_type=jax.ShapeDtypeStruct((B, D), jnp.float32),
        scratch_types=[
            pltpu.VMEM((b_per_w,), jnp.int32),
            pltpu.VMEM((_NBANK, _K, D, 128), jnp.float32),
            pltpu.VMEM((_HALF, D), jnp.float32),
            pltpu.SemaphoreType.DMA,
            pltpu.SemaphoreType.DMA,
            pltpu.SemaphoreType.DMA,
        ],
        compiler_params=pltpu.CompilerParams(needs_layout_passes=False),
    )
    def k(table_hbm, idx_hbm, out_hbm, idx_v, ring, out_v, sem0, sem1, sem2):
        wid = lax.axis_index("s") * NC + lax.axis_index("c")
        base = wid * b_per_w
        pltpu.sync_copy(idx_hbm.at[pl.ds(base, b_per_w)], idx_v)

        lane = lax.iota(jnp.int32, 16)
        neg_inf = jnp.int32(-2147483648)
        sems = [sem0, sem1, sem2]
        n_chunks_total = n_half * n_chunk

        def scalar_at(ci, j):
            iv16 = idx_v[pl.ds(ci * _K, _K)]
            return jnp.max(jnp.where(lane == j, iv16, neg_inf))

        def fire(ci, bank):
            def body(j, _):
                r = scalar_at(ci, j)
                blk = pl.multiple_of(
                    jnp.bitwise_and(r, jnp.int32(~127)), 128
                )
                pltpu.async_copy(
                    table_hbm.at[:, pl.ds(blk, 128)],
                    ring.at[bank, j],
                    sems[bank],
                )
                return 0

            lax.fori_loop(0, _K, body, 0)

        def drain_extract(ci, lc, bank):
            def wait_body(j, _):
                pltpu.make_async_copy(
                    table_hbm.at[:, pl.ds(0, 128)],
                    ring.at[bank, j],
                    sems[bank],
                ).wait()
                return 0

            lax.fori_loop(0, _K, wait_body, 0)

            def ext_body(j, _):
                r = scalar_at(ci, j)
                sub = jnp.bitwise_and(r, jnp.int32(127))
                val = plsc.load_gather(
                    ring.at[bank, j], [lane, jnp.full((16,), sub, jnp.int32)]
                )
                row = lc * _K + j
                plsc.store_scatter(
                    out_v, [jnp.full((16,), row, jnp.int32), lane], val
                )
                return 0

            lax.fori_loop(0, _K, ext_body, 0)

        fire(jnp.int32(0), 0)
        fire(jnp.int32(1), 1 % _NBANK)
        for ci in range(n_chunks_total):
            if ci + 2 < n_chunks_total:
                fire(jnp.int32(ci + 2), (ci + 2) % _NBANK)
            drain_extract(jnp.int32(ci), jnp.int32(ci % n_chunk), ci % _NBANK)
            if (ci + 1) % n_chunk == 0:
                half = ci // n_chunk
                pltpu.sync_copy(
                    out_v, out_hbm.at[pl.ds(base + half * _HALF, _HALF)]
                )

    return k


def _tc_gather_call(V, D, B):
    n_steps = B // _TC_BLK

    def body(idx_s, table_ref, out_ref, slab, sem):
        g = pl.program_id(0)
        base = g * _TC_BLK
        for i in range(_TC_BLK):
            r = idx_s[base + i]
            blk = pl.multiple_of(jnp.bitwise_and(r, jnp.int32(~127)), 128)
            pltpu.make_async_copy(
                table_ref.at[:, pl.ds(blk, 128)], slab.at[i], sem
            ).start()
        for i in range(_TC_BLK):
            pltpu.make_async_copy(
                table_ref.at[:, pl.ds(0, 128)], slab.at[i], sem
            ).wait()
        lanes = lax.broadcasted_iota(jnp.int32, (D, 128), 1)
        for q in range(_TC_BLK // 128):
            acc = jnp.zeros((D, 128), jnp.float32)
            for li in range(128):
                i = q * 128 + li
                r = idx_s[base + i]
                sub = jnp.bitwise_and(r, jnp.int32(127))
                rolled = pltpu.roll(slab[i], jnp.int32(li) - sub, axis=1)
                acc = jnp.where(lanes == li, rolled, acc)
            out_ref[:, q * 128 : (q + 1) * 128] = acc

    grid_spec = pltpu.PrefetchScalarGridSpec(
        num_scalar_prefetch=1,
        grid=(n_steps,),
        in_specs=[pl.BlockSpec(memory_space=pl.ANY)],
        out_specs=pl.BlockSpec((D, _TC_BLK), lambda g, idx_s: (0, g)),
        scratch_shapes=[
            pltpu.VMEM((_TC_BLK, D, 128), jnp.float32),
            pltpu.SemaphoreType.DMA,
        ],
    )
    return pl.pallas_call(
        body,
        grid_spec=grid_spec,
        out_shape=jax.ShapeDtypeStruct((D, B), jnp.float32),
    )


def kernel(data, channelindex):
    V, D = data.shape
    (B,) = channelindex.shape
    table_t = data.T
    idx = channelindex.astype(jnp.int32)
    out_sc = _sc_gather_call(V, D, _SC_B)(table_t, idx[:_SC_B])
    out_tc_t = _tc_gather_call(V, D, B - _SC_B)(idx[_SC_B:], table_t)
    return jnp.concatenate([out_sc, out_tc_t.T], axis=0)
